# baseline (speedup 1.0000x reference)
import functools

import jax
import jax.numpy as jnp
from jax import lax
from jax.experimental import pallas as pl
from jax.experimental.pallas import tpu as pltpu
from jax.experimental.pallas import tpu_sc as plsc

_N_EMBED = 8192
_CODE_DIM = 256
_BETA = 0.25
_ROWS = 8192
_TILE = 512
_NUM_TILES = _ROWS // _TILE


def _vq_tc_kernel(z_ref, w_ref, idx_ref, enc_ref, zq_ref, loss_ref, perp_ref,
                  counts_acc, loss_acc, wsq_acc):
    i = pl.program_id(0)
    z = z_ref[...]
    w = w_ref[...]

    @pl.when(i == 0)
    def _wsq():
        wsq_acc[...] = jnp.sum(w * w, axis=1)[None, :]

    z2 = -2.0 * z
    zw2 = lax.dot_general(z2, w, (((1,), (1,)), ((), ())),
                          preferred_element_type=jnp.float32)
    scores = zw2 + wsq_acc[...]

    idx = jnp.argmin(scores, axis=1).astype(jnp.int32)
    idx_ref[0, 0, :] = idx
    dmin = jnp.min(scores, axis=1)

    col = lax.broadcasted_iota(jnp.int32, (1, _N_EMBED), 1)
    onehot = (col == idx[:, None]).astype(jnp.float32)
    enc_ref[...] = onehot

    zq_ref[...] = lax.dot_general(onehot, w, (((1,), (0,)), ((), ())),
                                  preferred_element_type=jnp.float32)

    ones = jnp.ones((1, _TILE), jnp.float32)
    part_counts = lax.dot_general(ones, onehot, (((1,), (0,)), ((), ())),
                                  preferred_element_type=jnp.float32)
    part_loss = jnp.sum(dmin) + jnp.sum(z * z)

    @pl.when(i == 0)
    def _init():
        counts_acc[...] = part_counts
        loss_acc[0, 0] = part_loss

    @pl.when(i > 0)
    def _accum():
        counts_acc[...] += part_counts
        loss_acc[0, 0] += part_loss

    @pl.when(i == _NUM_TILES - 1)
    def _finish():
        loss = _BETA * loss_acc[0, 0] / (_ROWS * _CODE_DIM)
        loss_ref[...] = jnp.reshape(loss, (1, 1))
        avg = counts_acc[...] / _ROWS
        ent = jnp.sum(avg * jnp.log(avg + 1e-10))
        perp_ref[...] = jnp.reshape(jnp.exp(-ent), (1, 1))


def _run_tc(z_flat, weight):
    out_shapes = (
        jax.ShapeDtypeStruct((_NUM_TILES, 1, _TILE), jnp.int32),
        jax.ShapeDtypeStruct((_ROWS, _N_EMBED), jnp.float32),
        jax.ShapeDtypeStruct((_ROWS, _CODE_DIM), jnp.float32),
        jax.ShapeDtypeStruct((1, 1), jnp.float32),
        jax.ShapeDtypeStruct((1, 1), jnp.float32),
    )
    return pl.pallas_call(
        _vq_tc_kernel,
        grid=(_NUM_TILES,),
        in_specs=[
            pl.BlockSpec((_TILE, _CODE_DIM), lambda i: (i, 0)),
            pl.BlockSpec((_N_EMBED, _CODE_DIM), lambda i: (0, 0)),
        ],
        out_specs=(
            pl.BlockSpec((1, 1, _TILE), lambda i: (i, 0, 0)),
            pl.BlockSpec((_TILE, _N_EMBED), lambda i: (i, 0)),
            pl.BlockSpec((_TILE, _CODE_DIM), lambda i: (i, 0)),
            pl.BlockSpec((1, 1), lambda i: (0, 0)),
            pl.BlockSpec((1, 1), lambda i: (0, 0)),
        ),
        out_shape=out_shapes,
        scratch_shapes=[
            pltpu.VMEM((1, _N_EMBED), jnp.float32),
            pltpu.SMEM((1, 1), jnp.float32),
            pltpu.VMEM((1, _N_EMBED), jnp.float32),
        ],
    )(z_flat, weight)


def _make_sc_gather():
    info = plsc.get_sparse_core_info()
    nw = info.num_cores * info.num_subcores
    b_per_w = _ROWS // nw
    mesh = plsc.VectorSubcoreMesh(core_axis_name="c", subcore_axis_name="s")

    @functools.partial(
        pl.kernel, mesh=mesh,
        out_type=jax.ShapeDtypeStruct((_ROWS, _CODE_DIM), jnp.float32),
        scratch_types=[
            pltpu.VMEM((b_per_w,), jnp.int32),
            pltpu.VMEM((b_per_w, _CODE_DIM), jnp.float32),
            pltpu.SemaphoreType.DMA,
        ],
    )
    def gather_k(table_hbm, idx_hbm, out_hbm, idx_v, rows_v, sem):
        wid = lax.axis_index("s") * info.num_cores + lax.axis_index("c")
        base = wid * b_per_w
        pltpu.sync_copy(idx_hbm.at[pl.ds(base, b_per_w)], idx_v)
        pltpu.async_copy(table_hbm.at[idx_v], rows_v, sem).wait()
        pltpu.sync_copy(rows_v, out_hbm.at[pl.ds(base, b_per_w)])

    return gather_k


def kernel(z, weight):
    b, c, h, w = z.shape
    zt = jnp.transpose(z, (0, 2, 3, 1))
    z_flat = zt.reshape(-1, c)

    idx, encodings, zq_flat, loss, perp = _run_tc(z_flat, weight)
    idx_flat = idx.reshape(_ROWS)

    z_q_out = jnp.transpose(zq_flat.reshape(b, h, w, c), (0, 3, 1, 2))
    encoding_indices = idx_flat.reshape(b, h, w)
    return (z_q_out, loss[0, 0], perp[0, 0], encodings,
            encoding_indices)


# speedup vs baseline: 1.1374x; 1.1374x over previous
import functools

import jax
import jax.numpy as jnp
from jax import lax
from jax.experimental import pallas as pl
from jax.experimental.pallas import tpu as pltpu
from jax.experimental.pallas import tpu_sc as plsc

_N_EMBED = 8192
_CODE_DIM = 256
_BETA = 0.25
_ROWS = 8192
_TILE = 512
_NUM_TILES = _ROWS // _TILE


def _vq_tc_kernel(z_ref, w_ref, idx_ref, enc_ref, loss_ref, perp_ref,
                  counts_acc, loss_acc, wsq_acc):
    i = pl.program_id(0)
    z = z_ref[...]
    w = w_ref[...]

    @pl.when(i == 0)
    def _wsq():
        wsq_acc[...] = jnp.sum(w * w, axis=1)[None, :]

    z2 = -2.0 * z
    zw2 = lax.dot_general(z2, w, (((1,), (1,)), ((), ())),
                          preferred_element_type=jnp.float32)
    scores = zw2 + wsq_acc[...]

    idx = jnp.argmin(scores, axis=1).astype(jnp.int32)
    idx_ref[0, 0, :] = idx
    dmin = jnp.min(scores, axis=1)

    col = lax.broadcasted_iota(jnp.int32, (1, _N_EMBED), 1)
    onehot = (col == idx[:, None]).astype(jnp.float32)
    enc_ref[...] = onehot

    ones = jnp.ones((1, _TILE), jnp.float32)
    part_counts = lax.dot_general(ones, onehot, (((1,), (0,)), ((), ())),
                                  preferred_element_type=jnp.float32)
    part_loss = jnp.sum(dmin) + jnp.sum(z * z)

    @pl.when(i == 0)
    def _init():
        counts_acc[...] = part_counts
        loss_acc[0, 0] = part_loss

    @pl.when(i > 0)
    def _accum():
        counts_acc[...] += part_counts
        loss_acc[0, 0] += part_loss

    @pl.when(i == _NUM_TILES - 1)
    def _finish():
        loss = _BETA * loss_acc[0, 0] / (_ROWS * _CODE_DIM)
        loss_ref[...] = jnp.reshape(loss, (1, 1))
        avg = counts_acc[...] / _ROWS
        ent = jnp.sum(avg * jnp.log(avg + 1e-10))
        perp_ref[...] = jnp.reshape(jnp.exp(-ent), (1, 1))


def _run_tc(z_flat, weight):
    out_shapes = (
        jax.ShapeDtypeStruct((_NUM_TILES, 1, _TILE), jnp.int32),
        jax.ShapeDtypeStruct((_ROWS, _N_EMBED), jnp.float32),
        jax.ShapeDtypeStruct((1, 1), jnp.float32),
        jax.ShapeDtypeStruct((1, 1), jnp.float32),
    )
    return pl.pallas_call(
        _vq_tc_kernel,
        grid=(_NUM_TILES,),
        in_specs=[
            pl.BlockSpec((_TILE, _CODE_DIM), lambda i: (i, 0)),
            pl.BlockSpec((_N_EMBED, _CODE_DIM), lambda i: (0, 0)),
        ],
        out_specs=(
            pl.BlockSpec((1, 1, _TILE), lambda i: (i, 0, 0)),
            pl.BlockSpec((_TILE, _N_EMBED), lambda i: (i, 0)),
            pl.BlockSpec((1, 1), lambda i: (0, 0)),
            pl.BlockSpec((1, 1), lambda i: (0, 0)),
        ),
        out_shape=out_shapes,
        scratch_shapes=[
            pltpu.VMEM((1, _N_EMBED), jnp.float32),
            pltpu.SMEM((1, 1), jnp.float32),
            pltpu.VMEM((1, _N_EMBED), jnp.float32),
        ],
    )(z_flat, weight)


def _make_sc_gather():
    info = plsc.get_sparse_core_info()
    nw = info.num_cores * info.num_subcores
    b_per_w = _ROWS // nw
    mesh = plsc.VectorSubcoreMesh(core_axis_name="c", subcore_axis_name="s")

    @functools.partial(
        pl.kernel, mesh=mesh,
        out_type=jax.ShapeDtypeStruct((_ROWS, _CODE_DIM), jnp.float32),
        scratch_types=[
            pltpu.VMEM((b_per_w,), jnp.int32),
            pltpu.VMEM((b_per_w, _CODE_DIM), jnp.float32),
            pltpu.SemaphoreType.DMA,
        ],
    )
    def gather_k(table_hbm, idx_hbm, out_hbm, idx_v, rows_v, sem):
        wid = lax.axis_index("s") * info.num_cores + lax.axis_index("c")
        base = wid * b_per_w
        pltpu.sync_copy(idx_hbm.at[pl.ds(base, b_per_w)], idx_v)
        pltpu.async_copy(table_hbm.at[idx_v], rows_v, sem).wait()
        pltpu.sync_copy(rows_v, out_hbm.at[pl.ds(base, b_per_w)])

    return gather_k


def kernel(z, weight):
    b, c, h, w = z.shape
    zt = jnp.transpose(z, (0, 2, 3, 1))
    z_flat = zt.reshape(-1, c)

    idx, encodings, loss, perp = _run_tc(z_flat, weight)
    idx_flat = idx.reshape(_ROWS)

    zq_flat = _make_sc_gather()(weight, idx_flat)

    z_q_out = jnp.transpose(zq_flat.reshape(b, h, w, c), (0, 3, 1, 2))
    encoding_indices = idx_flat.reshape(b, h, w)
    return (z_q_out, loss[0, 0], perp[0, 0], encodings,
            encoding_indices)
